# baseline (speedup 1.0000x reference)
import functools

import jax
import jax.numpy as jnp
from jax import lax
from jax.experimental import pallas as pl
from jax.experimental.pallas import tpu as pltpu
from jax.experimental.pallas import tpu_sc as plsc

B = 1024
L = 200
DIM = 64
N = B * L
NW = 32
PER_W = N // NW
CHUNK = 128
NCHUNK = PER_W // CHUNK


def _gather_body(ids_ref, table_ref, out_ref, idx_v, rows_v, sem):
    wid = lax.axis_index("s") * 2 + lax.axis_index("c")
    chunk0 = wid * NCHUNK
    pltpu.sync_copy(ids_ref.at[wid], idx_v)

    def step(c, _):
        pltpu.async_copy(table_ref.at[idx_v.at[c]], rows_v, sem).wait()
        pltpu.sync_copy(rows_v, out_ref.at[pl.ds((chunk0 + c) * CHUNK, CHUNK)])
        return _

    lax.fori_loop(0, NCHUNK, step, 0)


@functools.partial(jax.jit, donate_argnums=())
def _sc_gather(ids2d, table):
    mesh = plsc.VectorSubcoreMesh(core_axis_name="c", subcore_axis_name="s")
    return pl.kernel(
        _gather_body,
        out_type=jax.ShapeDtypeStruct((N, DIM), jnp.float32),
        mesh=mesh,
        scratch_types=[
            pltpu.VMEM((NCHUNK, CHUNK), jnp.int32),
            pltpu.VMEM((CHUNK, DIM), jnp.float32),
            pltpu.SemaphoreType.DMA,
        ],
        compiler_params=pltpu.CompilerParams(use_tc_tiling_on_sc=False),
    )(ids2d, table)


def kernel(input_ids, attention_mask, table):
    ids2d = input_ids.reshape(NW, NCHUNK, CHUNK).astype(jnp.int32)
    flat = _sc_gather(ids2d, table)
    return (flat.reshape(B, L, DIM), attention_mask)


# speedup vs baseline: 1.0434x; 1.0434x over previous
import functools

import jax
import jax.numpy as jnp
from jax import lax
from jax.experimental import pallas as pl
from jax.experimental.pallas import tpu as pltpu
from jax.experimental.pallas import tpu_sc as plsc

B = 1024
L = 200
DIM = 64
N = B * L
NW = 32
PER_W = N // NW
CHUNK = 128
NCHUNK = PER_W // CHUNK
NBUF = 5
NT = NCHUNK // NBUF


def _gather_body(ids_ref, table_ref, out_ref, idx_v, bufs, gsem, wsem):
    wid = lax.axis_index("s") * 2 + lax.axis_index("c")
    chunk0 = wid * NCHUNK
    pltpu.sync_copy(ids_ref.at[wid], idx_v)

    def fire_gather(c, b):
        pltpu.async_copy(table_ref.at[idx_v.at[c]], bufs.at[b], gsem.at[b])

    def fire_write(c, b):
        pltpu.async_copy(
            bufs.at[b], out_ref.at[pl.ds((chunk0 + c) * CHUNK, CHUNK)],
            wsem.at[b])

    def wait_gather(b):
        pltpu.make_async_copy(
            table_ref.at[idx_v.at[0]], bufs.at[b], gsem.at[b]).wait()

    def wait_write(b):
        pltpu.make_async_copy(
            bufs.at[b], out_ref.at[pl.ds(0, CHUNK)], wsem.at[b]).wait()

    for b in range(NBUF):
        fire_gather(b, b)

    def turn(t, _):
        for b in range(NBUF):
            c = t * NBUF + b
            wait_gather(b)
            fire_write(c, b)
            wait_write(b)
            fire_gather(c + NBUF, b)
        return _

    lax.fori_loop(0, NT - 1, turn, 0)

    for b in range(NBUF):
        c = (NT - 1) * NBUF + b
        wait_gather(b)
        fire_write(c, b)
    for b in range(NBUF):
        wait_write(b)


@jax.jit
def _sc_gather(ids3d, table):
    mesh = plsc.VectorSubcoreMesh(core_axis_name="c", subcore_axis_name="s")
    return pl.kernel(
        _gather_body,
        out_type=jax.ShapeDtypeStruct((N, DIM), jnp.float32),
        mesh=mesh,
        scratch_types=[
            pltpu.VMEM((NCHUNK, CHUNK), jnp.int32),
            pltpu.VMEM((NBUF, CHUNK, DIM), jnp.float32),
            pltpu.SemaphoreType.DMA((NBUF,)),
            pltpu.SemaphoreType.DMA((NBUF,)),
        ],
        compiler_params=pltpu.CompilerParams(use_tc_tiling_on_sc=False),
    )(ids3d, table)


def kernel(input_ids, attention_mask, table):
    ids3d = input_ids.reshape(NW, NCHUNK, CHUNK).astype(jnp.int32)
    flat = _sc_gather(ids3d, table)
    return (flat.reshape(B, L, DIM), attention_mask)
